# baseline (speedup 1.0000x reference)
import functools
import math

import jax
import jax.numpy as jnp
from jax import lax
from jax.experimental import pallas as pl
from jax.experimental.pallas import tpu as pltpu
from jax.experimental.pallas import tpu_sc as plsc

_P = 0.2
_K = 2
_NW = 32
_CHUNK = 1024
_LANES = 16


def _build_sc_call(BC, NCH, RPW):
    mesh = plsc.VectorSubcoreMesh(core_axis_name="c", subcore_axis_name="s")
    GR = RPW * _K

    @functools.partial(
        pl.kernel,
        out_type=jax.ShapeDtypeStruct((BC, NCH, _CHUNK), jnp.float32),
        mesh=mesh,
        scratch_types=[
            pltpu.VMEM((GR,), jnp.int32),
            pltpu.VMEM((GR,), jnp.int32),
            pltpu.VMEM((RPW, _K, _LANES), jnp.float32),
            pltpu.VMEM((GR, _CHUNK), jnp.float32),
            pltpu.VMEM((RPW, 1, _CHUNK), jnp.float32),
            pltpu.SemaphoreType.DMA,
        ],
    )
    def sc_kernel(xr_hbm, rows_hbm, coef_hbm, out_hbm,
                  rows_v, idx_v, coef_v, g_v, o_v, sem):
        w = lax.axis_index("s") * 2 + lax.axis_index("c")
        base = w * RPW
        pltpu.sync_copy(rows_hbm.at[pl.ds(base * _K, GR)], rows_v)
        pltpu.sync_copy(coef_hbm.at[pl.ds(base, RPW)], coef_v)

        def chunk_body(j, carry):
            for t in range(GR // _LANES):
                sl = pl.ds(t * _LANES, _LANES)
                idx_v[sl] = rows_v[sl] * NCH + j
            pltpu.async_copy(xr_hbm.at[idx_v], g_v, sem).wait()

            def row_body(r, rc):
                a0 = coef_v[r, 0, :]
                a1 = coef_v[r, 1, :]

                def s_body(s, sc_):
                    for u in range(4):
                        sl = pl.ds((s * 4 + u) * _LANES, _LANES)
                        o_v[r, 0, sl] = a0 * g_v[2 * r, sl] + a1 * g_v[2 * r + 1, sl]
                    return sc_

                lax.fori_loop(0, _CHUNK // (4 * _LANES), s_body, 0)
                return rc

            lax.fori_loop(0, RPW, row_body, 0)
            pltpu.sync_copy(o_v, out_hbm.at[pl.ds(base, RPW), pl.ds(j, 1)])
            return carry

        lax.fori_loop(0, NCH, chunk_body, 0)

    return sc_kernel


def kernel(x, m_idx, m_w):
    B, C, H, W = x.shape
    K = m_idx.shape[1]
    BC = B * C
    HW = H * W
    NCH = HW // _CHUNK
    RPW = BC // _NW

    keep = jax.random.uniform(jax.random.key(1), (B, C)) > _P

    m_idx32 = m_idx.astype(jnp.int32)
    rself = (jnp.arange(B, dtype=jnp.int32)[:, None] * C
             + jnp.arange(C, dtype=jnp.int32)[None, :])
    src = jnp.arange(B, dtype=jnp.int32)[:, None, None] * C + m_idx32[None]
    rows = jnp.where(keep[:, :, None], rself[:, :, None], src)

    ckeep = jnp.array([1.0 / (1.0 - _P), 0.0], dtype=jnp.float32)
    cdrop = (m_w / (_P + 1e-12)).astype(jnp.float32)
    coef = jnp.where(keep[:, :, None], ckeep[None, None, :], cdrop[None])
    coefb = jnp.broadcast_to(
        coef.reshape(BC, K, 1), (BC, K, _LANES)).astype(jnp.float32)

    xr = x.reshape(BC * NCH, _CHUNK)
    out = _build_sc_call(BC, NCH, RPW)(
        xr, rows.reshape(BC * K), coefb)
    return out.reshape(B, C, H, W)


# speedup vs baseline: 1.2260x; 1.2260x over previous
import functools
import math

import jax
import jax.numpy as jnp
from jax import lax
from jax.experimental import pallas as pl
from jax.experimental.pallas import tpu as pltpu
from jax.experimental.pallas import tpu_sc as plsc

_P = 0.2
_K = 2
_NW = 32
_CHUNK = 512
_LANES = 16


def _build_sc_call(BC, NCH, RPW):
    mesh = plsc.VectorSubcoreMesh(core_axis_name="c", subcore_axis_name="s")
    GR = RPW * _K
    NIT = NCH // 2

    @functools.partial(
        pl.kernel,
        out_type=jax.ShapeDtypeStruct((BC, NCH, _CHUNK), jnp.float32),
        mesh=mesh,
        scratch_types=[
            pltpu.VMEM((GR,), jnp.int32),
            pltpu.VMEM((GR,), jnp.int32),
            pltpu.VMEM((GR,), jnp.int32),
            pltpu.VMEM((RPW, _K, _LANES), jnp.float32),
            pltpu.VMEM((GR, _CHUNK), jnp.float32),
            pltpu.VMEM((GR, _CHUNK), jnp.float32),
            pltpu.VMEM((RPW, 1, _CHUNK), jnp.float32),
            pltpu.VMEM((RPW, 1, _CHUNK), jnp.float32),
            pltpu.SemaphoreType.DMA,
            pltpu.SemaphoreType.DMA,
            pltpu.SemaphoreType.DMA,
            pltpu.SemaphoreType.DMA,
        ],
    )
    def sc_kernel(xr_hbm, rows_hbm, coef_hbm, out_hbm,
                  rows_v, idx0_v, idx1_v, coef_v, g0_v, g1_v, o0_v, o1_v,
                  sem_g0, sem_g1, sem_s0, sem_s1):
        w = lax.axis_index("s") * 2 + lax.axis_index("c")
        base = w * RPW
        pltpu.sync_copy(rows_hbm.at[pl.ds(base * _K, GR)], rows_v)
        pltpu.sync_copy(coef_hbm.at[pl.ds(base, RPW)], coef_v)

        def set_idx(idx_ref, j):
            for t in range(GR // _LANES):
                sl = pl.ds(t * _LANES, _LANES)
                idx_ref[sl] = rows_v[sl] * NCH + j

        def gather(idx_ref, g_ref, sem):
            return pltpu.make_async_copy(xr_hbm.at[idx_ref], g_ref, sem)

        def scatter(o_ref, j, sem):
            return pltpu.make_async_copy(
                o_ref, out_hbm.at[pl.ds(base, RPW), pl.ds(j, 1)], sem)

        def compute(g_ref, o_ref):
            def row_body(r, rc):
                a0 = coef_v[r, 0, :]
                a1 = coef_v[r, 1, :]

                def s_body(s, sc_):
                    for u in range(4):
                        sl = pl.ds((s * 4 + u) * _LANES, _LANES)
                        o_ref[r, 0, sl] = (a0 * g_ref[2 * r, sl]
                                           + a1 * g_ref[2 * r + 1, sl])
                    return sc_

                lax.fori_loop(0, _CHUNK // (4 * _LANES), s_body, 0)
                return rc

            lax.fori_loop(0, RPW, row_body, 0)

        set_idx(idx0_v, 0)
        gather(idx0_v, g0_v, sem_g0).start()
        set_idx(idx1_v, 1)
        gather(idx1_v, g1_v, sem_g1).start()

        def body(i, carry):
            a = 2 * i
            b = a + 1

            gather(idx0_v, g0_v, sem_g0).wait()

            @pl.when(i > 0)
            def _():
                scatter(o0_v, a, sem_s0).wait()

            compute(g0_v, o0_v)
            scatter(o0_v, a, sem_s0).start()

            @pl.when(i < NIT - 1)
            def _():
                set_idx(idx0_v, a + 2)
                gather(idx0_v, g0_v, sem_g0).start()

            gather(idx1_v, g1_v, sem_g1).wait()

            @pl.when(i > 0)
            def _():
                scatter(o1_v, b, sem_s1).wait()

            compute(g1_v, o1_v)
            scatter(o1_v, b, sem_s1).start()

            @pl.when(i < NIT - 1)
            def _():
                set_idx(idx1_v, b + 2)
                gather(idx1_v, g1_v, sem_g1).start()

            return carry

        lax.fori_loop(0, NIT, body, 0)
        scatter(o0_v, 0, sem_s0).wait()
        scatter(o1_v, 1, sem_s1).wait()

    return sc_kernel


def kernel(x, m_idx, m_w):
    B, C, H, W = x.shape
    K = m_idx.shape[1]
    BC = B * C
    HW = H * W
    NCH = HW // _CHUNK
    RPW = BC // _NW

    keep = jax.random.uniform(jax.random.key(1), (B, C)) > _P

    m_idx32 = m_idx.astype(jnp.int32)
    rself = (jnp.arange(B, dtype=jnp.int32)[:, None] * C
             + jnp.arange(C, dtype=jnp.int32)[None, :])
    src = jnp.arange(B, dtype=jnp.int32)[:, None, None] * C + m_idx32[None]
    rows = jnp.where(keep[:, :, None], rself[:, :, None], src)

    ckeep = jnp.array([1.0 / (1.0 - _P), 0.0], dtype=jnp.float32)
    cdrop = (m_w / (_P + 1e-12)).astype(jnp.float32)
    coef = jnp.where(keep[:, :, None], ckeep[None, None, :], cdrop[None])
    coefb = jnp.broadcast_to(
        coef.reshape(BC, K, 1), (BC, K, _LANES)).astype(jnp.float32)

    xr = x.reshape(BC * NCH, _CHUNK)
    out = _build_sc_call(BC, NCH, RPW)(
        xr, rows.reshape(BC * K), coefb)
    return out.reshape(B, C, H, W)
